# baseline (speedup 1.0000x reference)
import jax
import jax.numpy as jnp
from jax import lax
from jax.experimental import pallas as pl
from jax.experimental.pallas import tpu as pltpu
from jax.experimental.pallas import tpu_sc as plsc

N_NODES = 10000
N_EDGES = 160000
F = 256
ALPHA = 0.2

N_PAD = 10240
TPR = N_PAD // 32
ROW_BLK = 512
E_PAD = 163840
WIN = 1024
NWIN = E_PAD // WIN
CAP = WIN + 144
K = 32


def _tc_body(x_ref, w_ref, a2_ref, wh_ref, sd_ref):
    xb = x_ref[...]
    whb = jnp.dot(xb, w_ref[...], preferred_element_type=jnp.float32)
    wh_ref[...] = whb
    sd_ref[...] = lax.dot_general(
        a2_ref[...], whb, (((1,), (1,)), ((), ())),
        preferred_element_type=jnp.float32)


def _sc_body(wh_hbm, s_hbm, d_hbm, src_hbm, dst_hbm, out_hbm,
             hacc, s_v, d_own, se_v, de_v, srcC, dlC, wK,
             rows, denloc, invden, sem):
    c = lax.axis_index("c")
    t = lax.axis_index("s")
    tg = c * 16 + t
    g0 = tg * TPR

    zv = jnp.zeros((16,), jnp.float32)
    zi = jnp.zeros((16,), jnp.int32)

    def zero_h(r, _):
        for k in range(F // 16):
            hacc[r, pl.ds(16 * k, 16)] = zv
        return 0
    lax.fori_loop(0, TPR, zero_h, 0)

    def zero_den(i, _):
        denloc[pl.ds(16 * i, 16)] = zv
        return 0
    lax.fori_loop(0, (TPR + 16) // 16, zero_den, 0)

    pltpu.sync_copy(s_hbm, s_v)
    pltpu.sync_copy(d_hbm.at[pl.ds(g0, TPR)], d_own.at[pl.ds(0, TPR)])
    d_own[pl.ds(TPR, 16)] = zv

    def run_chunks(off):
        nchunks = (off + (K - 1)) // K

        def chunk_body(j, _):
            base = j * K
            for r in range(K):
                sr = srcC[pl.ds(base + r, 16)][0]
                pltpu.make_async_copy(wh_hbm.at[sr], rows.at[r], sem).start()
            pltpu.make_async_copy(
                wh_hbm.at[pl.ds(0, K), :], rows, sem).wait()
            for m in range(K // 16):
                sv = srcC[pl.ds(base + 16 * m, 16)]
                dl = dlC[pl.ds(base + 16 * m, 16)]
                sval = plsc.load_gather(s_v, [sv])
                dval = plsc.load_gather(d_own, [dl])
                e = sval + dval
                e = jnp.where(e >= 0.0, e, ALPHA * e)
                wK[pl.ds(16 * m, 16)] = jnp.exp(e)

            def acc_row(r, _):
                w = jnp.full((16,), wK[pl.ds(r, 16)][0])
                dl_r = dlC[pl.ds(base + r, 16)][0]
                for k in range(F // 16):
                    hacc[dl_r, pl.ds(16 * k, 16)] = (
                        hacc[dl_r, pl.ds(16 * k, 16)]
                        + w * rows[r, pl.ds(16 * k, 16)])
                return 0
            lax.fori_loop(0, K, acc_row, 0)

            for m in range(K // 16):
                dl = dlC[pl.ds(base + 16 * m, 16)]
                plsc.addupdate_scatter(denloc, [dl], wK[pl.ds(16 * m, 16)])
            return 0

        lax.fori_loop(0, nchunks, chunk_body, 0)

    def window(wi, _):
        ebase = wi * WIN
        pltpu.sync_copy(src_hbm.at[pl.ds(ebase, WIN)], se_v)
        pltpu.sync_copy(dst_hbm.at[pl.ds(ebase, WIN)], de_v)

        def scan_body(i, off):
            sv = se_v[pl.ds(i * 16, 16)]
            dv = de_v[pl.ds(i * 16, 16)]
            dl = dv - g0
            msk = (dl >= 0) & (dl < TPR)
            plsc.store_compressed(srcC.at[pl.ds(off, 16)], sv, mask=msk)
            plsc.store_compressed(dlC.at[pl.ds(off, 16)], dl, mask=msk)
            return off + jnp.sum(jnp.where(msk, 1, 0))

        off = lax.fori_loop(0, WIN // 16, scan_body, 0)

        for m in range(3):
            srcC[pl.ds(off + 16 * m, 16)] = zi
            dlC[pl.ds(off + 16 * m, 16)] = zi + TPR

        run_chunks(off)
        return 0

    lax.fori_loop(0, NWIN, window, 0)

    def self_chunk(j, _):
        base = j * K
        pltpu.sync_copy(wh_hbm.at[pl.ds(g0 + base, K)], rows)
        for m in range(K // 16):
            sval = s_v[pl.ds(g0 + base + 16 * m, 16)]
            dval = d_own[pl.ds(base + 16 * m, 16)]
            e = sval + dval
            e = jnp.where(e >= 0.0, e, ALPHA * e)
            w = jnp.exp(e)
            wK[pl.ds(16 * m, 16)] = w
            loc = base + 16 * m
            denloc[pl.ds(loc, 16)] = denloc[pl.ds(loc, 16)] + w

        def acc_row(r, _):
            w = jnp.full((16,), wK[pl.ds(r, 16)][0])
            for k in range(F // 16):
                hacc[base + r, pl.ds(16 * k, 16)] = (
                    hacc[base + r, pl.ds(16 * k, 16)]
                    + w * rows[r, pl.ds(16 * k, 16)])
            return 0
        lax.fori_loop(0, K, acc_row, 0)
        return 0

    lax.fori_loop(0, TPR // K, self_chunk, 0)

    def recip(i, _):
        invden[pl.ds(16 * i, 16)] = 1.0 / denloc[pl.ds(16 * i, 16)]
        return 0
    lax.fori_loop(0, TPR // 16, recip, 0)

    def fin_row(r, _):
        inv = jnp.full((16,), invden[pl.ds(r, 16)][0])
        for k in range(F // 16):
            h = hacc[r, pl.ds(16 * k, 16)] * inv
            hacc[r, pl.ds(16 * k, 16)] = jnp.where(
                h > 0.0, h, jnp.exp(h) - 1.0)
        return 0
    lax.fori_loop(0, TPR, fin_row, 0)

    @pl.when(g0 + TPR <= N_NODES)
    def _():
        pltpu.sync_copy(hacc.at[pl.ds(0, TPR), :], out_hbm.at[pl.ds(g0, TPR)])

    @pl.when((g0 < N_NODES) & (g0 + TPR > N_NODES))
    def _():
        pltpu.sync_copy(hacc.at[pl.ds(0, N_NODES % TPR), :],
                        out_hbm.at[pl.ds(g0, N_NODES % TPR)])


@jax.jit
def kernel(x, edge_index, W, a):
    x_pad = jnp.pad(x, ((0, N_PAD - N_NODES), (0, 0)))
    a2 = a[:, 0].reshape(2, F)
    a8 = jnp.concatenate([a2, jnp.zeros((6, F), jnp.float32)], axis=0)

    wh, sd = pl.pallas_call(
        _tc_body,
        grid=(N_PAD // ROW_BLK,),
        in_specs=[
            pl.BlockSpec((ROW_BLK, F), lambda i: (i, 0)),
            pl.BlockSpec((F, F), lambda i: (0, 0)),
            pl.BlockSpec((8, F), lambda i: (0, 0)),
        ],
        out_specs=[
            pl.BlockSpec((ROW_BLK, F), lambda i: (i, 0)),
            pl.BlockSpec((8, ROW_BLK), lambda i: (0, i)),
        ],
        out_shape=[
            jax.ShapeDtypeStruct((N_PAD, F), jnp.float32),
            jax.ShapeDtypeStruct((8, N_PAD), jnp.float32),
        ],
    )(x_pad, W, a8)

    src = edge_index[0].astype(jnp.int32)
    dst = edge_index[1].astype(jnp.int32)
    src = jnp.concatenate([src, jnp.zeros((E_PAD - N_EDGES,), jnp.int32)])
    dst = jnp.concatenate(
        [dst, jnp.full((E_PAD - N_EDGES,), -1, jnp.int32)])

    sc = pl.kernel(
        _sc_body,
        out_type=jax.ShapeDtypeStruct((N_NODES, F), jnp.float32),
        mesh=plsc.VectorSubcoreMesh(core_axis_name="c", subcore_axis_name="s"),
        compiler_params=pltpu.CompilerParams(needs_layout_passes=False),
        scratch_types=[
            pltpu.VMEM((TPR + 16, F), jnp.float32),
            pltpu.VMEM((N_PAD,), jnp.float32),
            pltpu.VMEM((TPR + 16,), jnp.float32),
            pltpu.VMEM((WIN,), jnp.int32),
            pltpu.VMEM((WIN,), jnp.int32),
            pltpu.VMEM((CAP,), jnp.int32),
            pltpu.VMEM((CAP,), jnp.int32),
            pltpu.VMEM((K + 16,), jnp.float32),
            pltpu.VMEM((K, F), jnp.float32),
            pltpu.VMEM((TPR + 16,), jnp.float32),
            pltpu.VMEM((TPR + 16,), jnp.float32),
            pltpu.SemaphoreType.DMA,
        ],
    )
    return sc(wh, sd[0], sd[1], src, dst)


# speedup vs baseline: 1.0221x; 1.0221x over previous
import jax
import jax.numpy as jnp
from jax import lax
from jax.experimental import pallas as pl
from jax.experimental.pallas import tpu as pltpu
from jax.experimental.pallas import tpu_sc as plsc

N_NODES = 10000
N_EDGES = 160000
F = 256
ALPHA = 0.2

N_PAD = 10240
TPR = N_PAD // 32
ROW_BLK = 512
E_PAD = 163840
WIN = 1024
NWIN = E_PAD // WIN
CAP = WIN + 144
K = 32


def _tc_body(x_ref, w_ref, a2_ref, wh_ref, sd_ref):
    xb = x_ref[...]
    whb = jnp.dot(xb, w_ref[...], preferred_element_type=jnp.float32)
    wh_ref[...] = whb
    sd_ref[...] = lax.dot_general(
        a2_ref[...], whb, (((1,), (1,)), ((), ())),
        preferred_element_type=jnp.float32)


def _sc_body(wh_hbm, s_hbm, d_hbm, src_hbm, dst_hbm, out_hbm,
             hacc, s_v, d_own, se_v, de_v, srcC, dlC, wK,
             rows, denloc, invden, sem):
    c = lax.axis_index("c")
    t = lax.axis_index("s")
    tg = c * 16 + t
    g0 = tg * TPR

    zv = jnp.zeros((16,), jnp.float32)
    zi = jnp.zeros((16,), jnp.int32)

    def zero_h(r, _):
        for k in range(F // 16):
            hacc[r, pl.ds(16 * k, 16)] = zv
        return 0
    lax.fori_loop(0, TPR, zero_h, 0)

    def zero_den(i, _):
        denloc[pl.ds(16 * i, 16)] = zv
        return 0
    lax.fori_loop(0, (TPR + 16) // 16, zero_den, 0)

    pltpu.sync_copy(s_hbm, s_v)
    pltpu.sync_copy(d_hbm.at[pl.ds(g0, TPR)], d_own.at[pl.ds(0, TPR)])
    d_own[pl.ds(TPR, 16)] = zv

    lane = lax.iota(jnp.int32, 16)

    def acc_rows(base):
        def acc_row(r, _):
            rsplat = jnp.full((16,), r, jnp.int32)
            w = plsc.load_gather(wK, [rsplat])
            dl = plsc.load_gather(dlC, [rsplat + base])
            for k in range(F // 32):
                u = rows[r, pl.ds(16 * k, 16)]
                ab = plsc.bitcast(u, jnp.bfloat16)
                a, b = plsc.unpack(ab, format=plsc.PackFormat.INTERLEAVED)
                ca = lane + 16 * k
                plsc.addupdate_scatter(hacc, [dl, ca], w * a)
                plsc.addupdate_scatter(hacc, [dl, ca + 128], w * b)
            return 0
        lax.fori_loop(0, K, acc_row, 0)

    def run_chunks(off):
        nchunks = (off + (K - 1)) // K

        def chunk_body(j, _):
            base = j * K
            pltpu.sync_copy(wh_hbm.at[srcC.at[pl.ds(base, K)]], rows)
            for m in range(K // 16):
                sv = srcC[pl.ds(base + 16 * m, 16)]
                dl = dlC[pl.ds(base + 16 * m, 16)]
                sval = plsc.load_gather(s_v, [sv])
                dval = plsc.load_gather(d_own, [dl])
                e = sval + dval
                e = jnp.where(e >= 0.0, e, ALPHA * e)
                wK[pl.ds(16 * m, 16)] = jnp.exp(e)
                plsc.addupdate_scatter(denloc, [dl], wK[pl.ds(16 * m, 16)])
            acc_rows(base)
            return 0

        lax.fori_loop(0, nchunks, chunk_body, 0)

    def window(wi, _):
        ebase = wi * WIN
        pltpu.sync_copy(src_hbm.at[pl.ds(ebase, WIN)], se_v)
        pltpu.sync_copy(dst_hbm.at[pl.ds(ebase, WIN)], de_v)

        def scan_body(i, off):
            sv = se_v[pl.ds(i * 16, 16)]
            dv = de_v[pl.ds(i * 16, 16)]
            dl = dv - g0
            msk = (dl >= 0) & (dl < TPR)
            plsc.store_compressed(srcC.at[pl.ds(off, 16)], sv, mask=msk)
            plsc.store_compressed(dlC.at[pl.ds(off, 16)], dl, mask=msk)
            return off + jnp.sum(jnp.where(msk, 1, 0))

        off = lax.fori_loop(0, WIN // 16, scan_body, 0)

        for m in range(3):
            srcC[pl.ds(off + 16 * m, 16)] = zi
            dlC[pl.ds(off + 16 * m, 16)] = zi + TPR

        run_chunks(off)
        return 0

    lax.fori_loop(0, NWIN, window, 0)

    def self_chunk(j, _):
        base = j * K
        pltpu.sync_copy(wh_hbm.at[pl.ds(g0 + base, K)], rows)
        for m in range(K // 16):
            sval = s_v[pl.ds(g0 + base + 16 * m, 16)]
            dval = d_own[pl.ds(base + 16 * m, 16)]
            e = sval + dval
            e = jnp.where(e >= 0.0, e, ALPHA * e)
            w = jnp.exp(e)
            wK[pl.ds(16 * m, 16)] = w
            loc = base + 16 * m
            denloc[pl.ds(loc, 16)] = denloc[pl.ds(loc, 16)] + w
            dlC[pl.ds(16 * m, 16)] = loc + lane
        acc_rows(0)
        return 0

    lax.fori_loop(0, TPR // K, self_chunk, 0)

    def recip(i, _):
        invden[pl.ds(16 * i, 16)] = 1.0 / denloc[pl.ds(16 * i, 16)]
        return 0
    lax.fori_loop(0, TPR // 16, recip, 0)

    def fin_row(r, _):
        inv = jnp.full((16,), invden[pl.ds(r, 16)][0])
        for k in range(F // 16):
            h = hacc[r, pl.ds(16 * k, 16)] * inv
            hacc[r, pl.ds(16 * k, 16)] = jnp.where(
                h > 0.0, h, jnp.exp(h) - 1.0)
        return 0
    lax.fori_loop(0, TPR, fin_row, 0)

    @pl.when(g0 + TPR <= N_NODES)
    def _():
        pltpu.sync_copy(hacc.at[pl.ds(0, TPR), :], out_hbm.at[pl.ds(g0, TPR)])

    @pl.when((g0 < N_NODES) & (g0 + TPR > N_NODES))
    def _():
        pltpu.sync_copy(hacc.at[pl.ds(0, N_NODES % TPR), :],
                        out_hbm.at[pl.ds(g0, N_NODES % TPR)])


@jax.jit
def kernel(x, edge_index, W, a):
    x_pad = jnp.pad(x, ((0, N_PAD - N_NODES), (0, 0)))
    a2 = a[:, 0].reshape(2, F)
    a8 = jnp.concatenate([a2, jnp.zeros((6, F), jnp.float32)], axis=0)

    wh, sd = pl.pallas_call(
        _tc_body,
        grid=(N_PAD // ROW_BLK,),
        in_specs=[
            pl.BlockSpec((ROW_BLK, F), lambda i: (i, 0)),
            pl.BlockSpec((F, F), lambda i: (0, 0)),
            pl.BlockSpec((8, F), lambda i: (0, 0)),
        ],
        out_specs=[
            pl.BlockSpec((ROW_BLK, F), lambda i: (i, 0)),
            pl.BlockSpec((8, ROW_BLK), lambda i: (0, i)),
        ],
        out_shape=[
            jax.ShapeDtypeStruct((N_PAD, F), jnp.float32),
            jax.ShapeDtypeStruct((8, N_PAD), jnp.float32),
        ],
    )(x_pad, W, a8)

    whbf = wh.astype(jnp.bfloat16)
    whp = jax.lax.bitcast_convert_type(
        jnp.stack([whbf[:, :128], whbf[:, 128:]], axis=-1), jnp.int32)

    src = edge_index[0].astype(jnp.int32)
    dst = edge_index[1].astype(jnp.int32)
    src = jnp.concatenate([src, jnp.zeros((E_PAD - N_EDGES,), jnp.int32)])
    dst = jnp.concatenate(
        [dst, jnp.full((E_PAD - N_EDGES,), -1, jnp.int32)])

    sc = pl.kernel(
        _sc_body,
        out_type=jax.ShapeDtypeStruct((N_NODES, F), jnp.float32),
        mesh=plsc.VectorSubcoreMesh(core_axis_name="c", subcore_axis_name="s"),
        compiler_params=pltpu.CompilerParams(needs_layout_passes=False),
        scratch_types=[
            pltpu.VMEM((TPR + 16, F), jnp.float32),
            pltpu.VMEM((N_PAD,), jnp.float32),
            pltpu.VMEM((TPR + 16,), jnp.float32),
            pltpu.VMEM((WIN,), jnp.int32),
            pltpu.VMEM((WIN,), jnp.int32),
            pltpu.VMEM((CAP,), jnp.int32),
            pltpu.VMEM((CAP,), jnp.int32),
            pltpu.VMEM((K + 16,), jnp.float32),
            pltpu.VMEM((K, F // 2), jnp.int32),
            pltpu.VMEM((TPR + 16,), jnp.float32),
            pltpu.VMEM((TPR + 16,), jnp.float32),
            pltpu.SemaphoreType.DMA,
        ],
    )
    return sc(whp, sd[0], sd[1], src, dst)


# speedup vs baseline: 2.9642x; 2.9002x over previous
import jax
import jax.numpy as jnp
from jax import lax
from jax.experimental import pallas as pl
from jax.experimental.pallas import tpu as pltpu
from jax.experimental.pallas import tpu_sc as plsc

N_NODES = 10000
N_EDGES = 160000
F = 256
ALPHA = 0.2

N_PAD = 10240
TPR = N_PAD // 32
ROW_BLK = 512
E_PAD = 164352
WIN = 768
NWIN = E_PAD // WIN
CAPE = 6640
CHK = 128
NCHK = N_PAD // CHK
HW = F // 2


def _tc_body(x_ref, w_ref, a2_ref, wh_ref, sd_ref):
    xb = x_ref[...]
    whb = jnp.dot(xb, w_ref[...], preferred_element_type=jnp.float32)
    wh_ref[...] = whb
    sd_ref[...] = lax.dot_general(
        a2_ref[...], whb, (((1,), (1,)), ((), ())),
        preferred_element_type=jnp.float32)


def _sc_body(whp_hbm, s_hbm, d_hbm, src_hbm, dst_hbm, out_hbm,
             hacc, packedE, mlist, whb0, whb1, fbuf,
             sem0, sem1):
    DEN = 336
    SCH = 672
    WKO = 800
    IKO = CAPE + 16

    c = lax.axis_index("c")
    t = lax.axis_index("s")
    tg = c * 16 + t
    g0 = tg * TPR

    zv = jnp.zeros((16,), jnp.float32)
    lane = lax.iota(jnp.int32, 16)

    def zero_h(r, _):
        for k in range(F // 16):
            hacc[r, pl.ds(16 * k, 16)] = zv
        return 0
    lax.fori_loop(0, TPR + 1, zero_h, 0)

    def zero_den(i, _):
        fbuf[pl.ds(DEN + 16 * i, 16)] = zv
        return 0
    lax.fori_loop(0, 21, zero_den, 0)

    fbuf[pl.ds(TPR, 16)] = zv
    pltpu.sync_copy(d_hbm.at[pl.ds(g0, TPR)], fbuf.at[pl.ds(0, TPR)])

    def round_body(wstart):
        def self_fill(i, _):
            dl = 16 * i + lane
            packedE[pl.ds(16 * i, 16)] = ((g0 + dl) << 9) | dl
            return 0
        lax.fori_loop(0, TPR // 16, self_fill, 0)
        off0 = jnp.where(wstart == 0, TPR, 0)

        def scan_cond(cw):
            w, off = cw
            return (w < NWIN) & (off <= CAPE - WIN)

        def scan_window(cw):
            w, off = cw
            ebase = w * WIN
            pltpu.sync_copy(src_hbm.at[pl.ds(ebase, WIN)],
                            mlist.at[pl.ds(0, WIN)])
            pltpu.sync_copy(dst_hbm.at[pl.ds(ebase, WIN)],
                            mlist.at[pl.ds(WIN, WIN)])

            def scan_body(i, o):
                sv = mlist[pl.ds(i * 16, 16)]
                dv = mlist[pl.ds(WIN + i * 16, 16)]
                dl = dv - g0
                msk = (dl >= 0) & (dl < TPR)
                p = (sv << 9) | jnp.clip(dl, 0, TPR - 1)
                plsc.store_compressed(packedE.at[pl.ds(o, 16)], p, mask=msk)
                return o + jnp.sum(jnp.where(msk, 1, 0))

            off = lax.fori_loop(0, WIN // 16, scan_body, off)
            return w + 1, off

        wnext, off = lax.while_loop(scan_cond, scan_window, (wstart, off0))

        packedE[pl.ds(off, 16)] = jnp.full((16,), TPR, jnp.int32)
        ngrp = (off + 15) // 16

        def issue(cidx, buf, sem):
            pltpu.make_async_copy(
                whp_hbm.at[pl.ds(cidx * CHK, CHK), :], buf, sem).start()

        def wait(buf, sem):
            pltpu.make_async_copy(
                whp_hbm.at[pl.ds(0, CHK), :], buf, sem).wait()

        def process(cidx, buf):
            c0 = cidx * CHK
            pltpu.sync_copy(s_hbm.at[pl.ds(c0, CHK)],
                            fbuf.at[pl.ds(SCH, CHK)])

            def rescan(i, mo):
                p = packedE[pl.ds(i * 16, 16)]
                soff = lax.shift_right_logical(p, 9) - c0
                msk = (soff >= 0) & (soff < CHK)
                plsc.store_compressed(mlist.at[pl.ds(mo, 16)], p, mask=msk)
                return mo + jnp.sum(jnp.where(msk, 1, 0))

            moff = lax.fori_loop(0, ngrp, rescan, 0)
            mlist[pl.ds(moff, 16)] = jnp.full((16,), (c0 << 9) | TPR,
                                              jnp.int32)

            def group(gi, _):
                p = mlist[pl.ds(gi * 16, 16)]
                soff = lax.shift_right_logical(p, 9) - c0
                dl = p & (512 - 1)
                sval = plsc.load_gather(fbuf, [soff + SCH])
                dval = plsc.load_gather(fbuf, [dl])
                e = sval + dval
                e = jnp.where(e >= 0.0, e, ALPHA * e)
                w = jnp.exp(e)
                fbuf[pl.ds(WKO, 16)] = w
                plsc.addupdate_scatter(fbuf, [dl + DEN], w)
                mlist[pl.ds(IKO, 16)] = soff
                mlist[pl.ds(IKO + 16, 16)] = dl

                def edge(r, _):
                    rsplat = jnp.full((16,), r, jnp.int32)
                    wsp = plsc.load_gather(fbuf, [rsplat + WKO])
                    ssp = plsc.load_gather(mlist, [rsplat + IKO])
                    dsp = plsc.load_gather(mlist, [rsplat + IKO + 16])
                    for k in range(HW // 16):
                        u = plsc.load_gather(buf, [ssp, lane + 16 * k])
                        ab = plsc.bitcast(u, jnp.bfloat16)
                        a, b = plsc.unpack(
                            ab, format=plsc.PackFormat.INTERLEAVED)
                        ca = lane + 16 * k
                        plsc.addupdate_scatter(hacc, [dsp, ca], wsp * a)
                        plsc.addupdate_scatter(hacc, [dsp, ca + 128], wsp * b)
                    return 0

                lax.fori_loop(0, 16, edge, 0)
                return 0

            lax.fori_loop(0, (moff + 15) // 16, group, 0)

        issue(0, whb0, sem0)
        issue(1, whb1, sem1)

        def pair(pi, _):
            wait(whb0, sem0)
            process(2 * pi, whb0)

            @pl.when(2 * pi + 2 < NCHK)
            def _():
                issue(2 * pi + 2, whb0, sem0)

            wait(whb1, sem1)
            process(2 * pi + 1, whb1)

            @pl.when(2 * pi + 3 < NCHK)
            def _():
                issue(2 * pi + 3, whb1, sem1)
            return 0

        lax.fori_loop(0, NCHK // 2, pair, 0)
        return wnext

    lax.while_loop(lambda w: w < NWIN, round_body, 0)

    def recip(i, _):
        fbuf[pl.ds(DEN + 16 * i, 16)] = 1.0 / fbuf[pl.ds(DEN + 16 * i, 16)]
        return 0
    lax.fori_loop(0, TPR // 16, recip, 0)

    def fin_row(r, _):
        inv = plsc.load_gather(fbuf, [jnp.full((16,), DEN + r, jnp.int32)])
        for k in range(F // 16):
            h = hacc[r, pl.ds(16 * k, 16)] * inv
            hacc[r, pl.ds(16 * k, 16)] = jnp.where(
                h > 0.0, h, jnp.exp(h) - 1.0)
        return 0
    lax.fori_loop(0, TPR, fin_row, 0)

    @pl.when(g0 + TPR <= N_NODES)
    def _():
        pltpu.sync_copy(hacc.at[pl.ds(0, TPR), :], out_hbm.at[pl.ds(g0, TPR)])

    @pl.when((g0 < N_NODES) & (g0 + TPR > N_NODES))
    def _():
        pltpu.sync_copy(hacc.at[pl.ds(0, N_NODES % TPR), :],
                        out_hbm.at[pl.ds(g0, N_NODES % TPR)])


@jax.jit
def kernel(x, edge_index, W, a):
    x_pad = jnp.pad(x, ((0, N_PAD - N_NODES), (0, 0)))
    a2 = a[:, 0].reshape(2, F)
    a8 = jnp.concatenate([a2, jnp.zeros((6, F), jnp.float32)], axis=0)

    wh, sd = pl.pallas_call(
        _tc_body,
        grid=(N_PAD // ROW_BLK,),
        in_specs=[
            pl.BlockSpec((ROW_BLK, F), lambda i: (i, 0)),
            pl.BlockSpec((F, F), lambda i: (0, 0)),
            pl.BlockSpec((8, F), lambda i: (0, 0)),
        ],
        out_specs=[
            pl.BlockSpec((ROW_BLK, F), lambda i: (i, 0)),
            pl.BlockSpec((8, ROW_BLK), lambda i: (0, i)),
        ],
        out_shape=[
            jax.ShapeDtypeStruct((N_PAD, F), jnp.float32),
            jax.ShapeDtypeStruct((8, N_PAD), jnp.float32),
        ],
    )(x_pad, W, a8)

    whbf = wh.astype(jnp.bfloat16)
    whp = jax.lax.bitcast_convert_type(
        jnp.stack([whbf[:, :HW], whbf[:, HW:]], axis=-1), jnp.int32)

    src = edge_index[0].astype(jnp.int32)
    dst = edge_index[1].astype(jnp.int32)
    src = jnp.concatenate([src, jnp.zeros((E_PAD - N_EDGES,), jnp.int32)])
    dst = jnp.concatenate(
        [dst, jnp.full((E_PAD - N_EDGES,), -1, jnp.int32)])

    sc = pl.kernel(
        _sc_body,
        out_type=jax.ShapeDtypeStruct((N_NODES, F), jnp.float32),
        mesh=plsc.VectorSubcoreMesh(core_axis_name="c", subcore_axis_name="s"),
        compiler_params=pltpu.CompilerParams(needs_layout_passes=False),
        scratch_types=[
            pltpu.VMEM((TPR + 1, F), jnp.float32),
            pltpu.VMEM((CAPE + 16,), jnp.int32),
            pltpu.VMEM((CAPE + 48,), jnp.int32),
            pltpu.VMEM((CHK, HW), jnp.int32),
            pltpu.VMEM((CHK, HW), jnp.int32),
            pltpu.VMEM((816,), jnp.float32),
            pltpu.SemaphoreType.DMA,
            pltpu.SemaphoreType.DMA,
        ],
    )
    return sc(whp, sd[0], sd[1], src, dst)


# speedup vs baseline: 3.6904x; 1.2450x over previous
import jax
import jax.numpy as jnp
from jax import lax
from jax.experimental import pallas as pl
from jax.experimental.pallas import tpu as pltpu
from jax.experimental.pallas import tpu_sc as plsc

N_NODES = 10000
N_EDGES = 160000
F = 256
ALPHA = 0.2

N_PAD = 10240
TPR = N_PAD // 32
ROW_BLK = 512
E_PAD = 164352
WIN = 768
NWIN = E_PAD // WIN
CAPE = 6640
CHK = 128
NCHK = N_PAD // CHK
HW = F // 2


def _tc_body(x_ref, w_ref, a2_ref, wh_ref, sd_ref):
    xb = x_ref[...]
    whb = jnp.dot(xb, w_ref[...], preferred_element_type=jnp.float32)
    wh_ref[...] = whb
    sd_ref[...] = lax.dot_general(
        a2_ref[...], whb, (((1,), (1,)), ((), ())),
        preferred_element_type=jnp.float32)


def _sc_body(whp_hbm, s_hbm, d_hbm, src_hbm, dst_hbm, out_hbm,
             hacc, packedE, mlist, whb0, whb1, fbuf,
             sem0, sem1, sem2):
    DEN = 336
    SCH = 672
    WKO = 800
    IKO = CAPE + 16

    c = lax.axis_index("c")
    t = lax.axis_index("s")
    tg = c * 16 + t
    g0 = tg * TPR

    zv = jnp.zeros((16,), jnp.float32)
    lane = lax.iota(jnp.int32, 16)

    def zero_h(r, _):
        for k in range(F // 16):
            hacc[r, pl.ds(16 * k, 16)] = zv
        return 0
    lax.fori_loop(0, TPR + 1, zero_h, 0)

    def zero_den(i, _):
        fbuf[pl.ds(DEN + 16 * i, 16)] = zv
        return 0
    lax.fori_loop(0, 21, zero_den, 0)

    fbuf[pl.ds(TPR, 16)] = zv
    pltpu.sync_copy(d_hbm.at[pl.ds(g0, TPR)], fbuf.at[pl.ds(0, TPR)])

    def round_body(wstart):
        def self_fill(i, _):
            dl = 16 * i + lane
            packedE[pl.ds(16 * i, 16)] = ((g0 + dl) << 9) | dl
            return 0
        lax.fori_loop(0, TPR // 16, self_fill, 0)
        off0 = jnp.where(wstart == 0, TPR, 0)

        def stage(w, par):
            ebase = jnp.minimum(w, NWIN - 1) * WIN
            base = par * (2 * WIN)
            pltpu.make_async_copy(src_hbm.at[pl.ds(ebase, WIN)],
                                  mlist.at[pl.ds(base, WIN)], sem2).start()
            pltpu.make_async_copy(dst_hbm.at[pl.ds(ebase, WIN)],
                                  mlist.at[pl.ds(base + WIN, WIN)],
                                  sem2).start()

        def drain_stage():
            pltpu.make_async_copy(src_hbm.at[pl.ds(0, 2 * WIN)],
                                  mlist.at[pl.ds(0, 2 * WIN)], sem2).wait()

        def scan_cond(cw):
            w, off = cw
            return (w < NWIN) & (off <= CAPE - WIN)

        def scan_window(cw):
            w, off = cw
            drain_stage()
            stage(w + 1, (w + 1) & 1)
            base = (w & 1) * (2 * WIN)

            def scan_body(i, o):
                sv = mlist[pl.ds(base + i * 16, 16)]
                dv = mlist[pl.ds(base + WIN + i * 16, 16)]
                dl = dv - g0
                msk = (dl >= 0) & (dl < TPR)
                p = (sv << 9) | jnp.clip(dl, 0, TPR - 1)
                plsc.store_compressed(packedE.at[pl.ds(o, 16)], p, mask=msk)
                return o + jnp.sum(jnp.where(msk, 1, 0))

            off = lax.fori_loop(0, WIN // 16, scan_body, off)
            return w + 1, off

        stage(wstart, wstart & 1)
        wnext, off = lax.while_loop(scan_cond, scan_window, (wstart, off0))
        drain_stage()

        packedE[pl.ds(off, 16)] = jnp.full((16,), TPR, jnp.int32)
        ngrp = (off + 15) // 16

        def issue(cidx, buf, sem):
            pltpu.make_async_copy(
                whp_hbm.at[pl.ds(cidx * CHK, CHK), :], buf, sem).start()

        def wait(buf, sem):
            pltpu.make_async_copy(
                whp_hbm.at[pl.ds(0, CHK), :], buf, sem).wait()

        def process(cidx, buf):
            c0 = cidx * CHK
            pltpu.sync_copy(s_hbm.at[pl.ds(c0, CHK)],
                            fbuf.at[pl.ds(SCH, CHK)])

            def rescan(i, mo):
                p = packedE[pl.ds(i * 16, 16)]
                soff = lax.shift_right_logical(p, 9) - c0
                msk = (soff >= 0) & (soff < CHK)
                plsc.store_compressed(mlist.at[pl.ds(mo, 16)], p, mask=msk)
                return mo + jnp.sum(jnp.where(msk, 1, 0))

            moff = lax.fori_loop(0, ngrp, rescan, 0)
            mlist[pl.ds(moff, 16)] = jnp.full((16,), (c0 << 9) | TPR,
                                              jnp.int32)

            def group(gi, _):
                p = mlist[pl.ds(gi * 16, 16)]
                soff = lax.shift_right_logical(p, 9) - c0
                dl = p & (512 - 1)
                sval = plsc.load_gather(fbuf, [soff + SCH])
                dval = plsc.load_gather(fbuf, [dl])
                e = sval + dval
                e = jnp.where(e >= 0.0, e, ALPHA * e)
                w = jnp.exp(e)
                fbuf[pl.ds(WKO, 16)] = w
                plsc.addupdate_scatter(fbuf, [dl + DEN], w)
                mlist[pl.ds(IKO, 16)] = soff
                mlist[pl.ds(IKO + 16, 16)] = dl

                def edge(r, _):
                    rsplat = jnp.full((16,), r, jnp.int32)
                    wsp = plsc.load_gather(fbuf, [rsplat + WKO])
                    ssp = plsc.load_gather(mlist, [rsplat + IKO])
                    dsp = plsc.load_gather(mlist, [rsplat + IKO + 16])
                    for k in range(HW // 16):
                        u = plsc.load_gather(buf, [ssp, lane + 16 * k])
                        ab = plsc.bitcast(u, jnp.bfloat16)
                        a, b = plsc.unpack(
                            ab, format=plsc.PackFormat.INTERLEAVED)
                        ca = lane + 16 * k
                        plsc.addupdate_scatter(hacc, [dsp, ca], wsp * a)
                        plsc.addupdate_scatter(hacc, [dsp, ca + 128], wsp * b)
                    return 0

                lax.fori_loop(0, 16, edge, 0)
                return 0

            lax.fori_loop(0, (moff + 15) // 16, group, 0)

        issue(0, whb0, sem0)
        issue(1, whb1, sem1)

        def pair(pi, _):
            wait(whb0, sem0)
            process(2 * pi, whb0)

            @pl.when(2 * pi + 2 < NCHK)
            def _():
                issue(2 * pi + 2, whb0, sem0)

            wait(whb1, sem1)
            process(2 * pi + 1, whb1)

            @pl.when(2 * pi + 3 < NCHK)
            def _():
                issue(2 * pi + 3, whb1, sem1)
            return 0

        lax.fori_loop(0, NCHK // 2, pair, 0)
        return wnext

    lax.while_loop(lambda w: w < NWIN, round_body, 0)

    def recip(i, _):
        fbuf[pl.ds(DEN + 16 * i, 16)] = 1.0 / fbuf[pl.ds(DEN + 16 * i, 16)]
        return 0
    lax.fori_loop(0, TPR // 16, recip, 0)

    def fin_row(r, _):
        inv = plsc.load_gather(fbuf, [jnp.full((16,), DEN + r, jnp.int32)])
        for k in range(F // 16):
            h = hacc[r, pl.ds(16 * k, 16)] * inv
            hacc[r, pl.ds(16 * k, 16)] = jnp.where(
                h > 0.0, h, jnp.exp(h) - 1.0)
        return 0
    lax.fori_loop(0, TPR, fin_row, 0)

    @pl.when(g0 + TPR <= N_NODES)
    def _():
        pltpu.sync_copy(hacc.at[pl.ds(0, TPR), :], out_hbm.at[pl.ds(g0, TPR)])

    @pl.when((g0 < N_NODES) & (g0 + TPR > N_NODES))
    def _():
        pltpu.sync_copy(hacc.at[pl.ds(0, N_NODES % TPR), :],
                        out_hbm.at[pl.ds(g0, N_NODES % TPR)])


@jax.jit
def kernel(x, edge_index, W, a):
    x_pad = jnp.pad(x, ((0, N_PAD - N_NODES), (0, 0)))
    a2 = a[:, 0].reshape(2, F)
    a8 = jnp.concatenate([a2, jnp.zeros((6, F), jnp.float32)], axis=0)

    wh, sd = pl.pallas_call(
        _tc_body,
        grid=(N_PAD // ROW_BLK,),
        in_specs=[
            pl.BlockSpec((ROW_BLK, F), lambda i: (i, 0)),
            pl.BlockSpec((F, F), lambda i: (0, 0)),
            pl.BlockSpec((8, F), lambda i: (0, 0)),
        ],
        out_specs=[
            pl.BlockSpec((ROW_BLK, F), lambda i: (i, 0)),
            pl.BlockSpec((8, ROW_BLK), lambda i: (0, i)),
        ],
        out_shape=[
            jax.ShapeDtypeStruct((N_PAD, F), jnp.float32),
            jax.ShapeDtypeStruct((8, N_PAD), jnp.float32),
        ],
    )(x_pad, W, a8)

    whbf = wh.astype(jnp.bfloat16)
    whp = jax.lax.bitcast_convert_type(
        jnp.stack([whbf[:, :HW], whbf[:, HW:]], axis=-1), jnp.int32)

    src = edge_index[0].astype(jnp.int32)
    dst = edge_index[1].astype(jnp.int32)
    src = jnp.concatenate([src, jnp.zeros((E_PAD - N_EDGES,), jnp.int32)])
    dst = jnp.concatenate(
        [dst, jnp.full((E_PAD - N_EDGES,), -1, jnp.int32)])

    sc = pl.kernel(
        _sc_body,
        out_type=jax.ShapeDtypeStruct((N_NODES, F), jnp.float32),
        mesh=plsc.VectorSubcoreMesh(core_axis_name="c", subcore_axis_name="s"),
        compiler_params=pltpu.CompilerParams(needs_layout_passes=False),
        scratch_types=[
            pltpu.VMEM((TPR + 1, F), jnp.float32),
            pltpu.VMEM((CAPE + 16,), jnp.int32),
            pltpu.VMEM((CAPE + 48,), jnp.int32),
            pltpu.VMEM((CHK, HW), jnp.int32),
            pltpu.VMEM((CHK, HW), jnp.int32),
            pltpu.VMEM((816,), jnp.float32),
            pltpu.SemaphoreType.DMA,
            pltpu.SemaphoreType.DMA,
            pltpu.SemaphoreType.DMA,
        ],
    )
    return sc(whp, sd[0], sd[1], src, dst)


# speedup vs baseline: 4.8908x; 1.3253x over previous
import jax
import jax.numpy as jnp
from jax import lax
from jax.experimental import pallas as pl
from jax.experimental.pallas import tpu as pltpu
from jax.experimental.pallas import tpu_sc as plsc

N_NODES = 10000
N_EDGES = 160000
F = 256
ALPHA = 0.2

N_PAD = 10240
TPR = N_PAD // 32
ROW_BLK = 512
E_PAD = 164352
WIN = 768
NWIN = E_PAD // WIN
CAPE = 6640
CHK = 128
NCHK = N_PAD // CHK
HW = F // 2


def _tc_body(x_ref, w_ref, a2_ref, wh_ref, sd_ref):
    xb = x_ref[...]
    whb = jnp.dot(xb, w_ref[...], preferred_element_type=jnp.float32)
    wh_ref[...] = whb
    sd_ref[...] = lax.dot_general(
        a2_ref[...], whb, (((1,), (1,)), ((), ())),
        preferred_element_type=jnp.float32)


def _sc_body(whp_hbm, s_hbm, d_hbm, src_hbm, dst_hbm, out_hbm,
             hacc, packedE, mlist, whb0, whb1, fbuf,
             sem0, sem1, sem2):
    DEN = 336
    SCH = 672
    WKO = 800
    IKO = CAPE + 16

    c = lax.axis_index("c")
    t = lax.axis_index("s")
    tg = c * 16 + t
    g0 = tg * TPR

    zv = jnp.zeros((16,), jnp.float32)
    lane = lax.iota(jnp.int32, 16)

    def zero_h(r, _):
        for k in range(F // 16):
            hacc[r, pl.ds(16 * k, 16)] = zv
        return 0
    lax.fori_loop(0, TPR + 1, zero_h, 0)

    def zero_den(i, _):
        fbuf[pl.ds(DEN + 16 * i, 16)] = zv
        return 0
    lax.fori_loop(0, 21, zero_den, 0)

    fbuf[pl.ds(TPR, 16)] = zv
    pltpu.sync_copy(d_hbm.at[pl.ds(g0, TPR)], fbuf.at[pl.ds(0, TPR)])

    def round_body(wstart):
        def self_fill(i, _):
            dl = 16 * i + lane
            packedE[pl.ds(16 * i, 16)] = ((g0 + dl) << 9) | dl
            return 0
        lax.fori_loop(0, TPR // 16, self_fill, 0)
        off0 = jnp.where(wstart == 0, TPR, 0)

        def stage(w, par):
            ebase = jnp.minimum(w, NWIN - 1) * WIN
            base = par * (2 * WIN)
            pltpu.make_async_copy(src_hbm.at[pl.ds(ebase, WIN)],
                                  mlist.at[pl.ds(base, WIN)], sem2).start()
            pltpu.make_async_copy(dst_hbm.at[pl.ds(ebase, WIN)],
                                  mlist.at[pl.ds(base + WIN, WIN)],
                                  sem2).start()

        def drain_stage():
            pltpu.make_async_copy(src_hbm.at[pl.ds(0, 2 * WIN)],
                                  mlist.at[pl.ds(0, 2 * WIN)], sem2).wait()

        def scan_cond(cw):
            w, off = cw
            return (w < NWIN) & (off <= CAPE - WIN)

        def scan_window(cw):
            w, off = cw
            drain_stage()
            stage(w + 1, (w + 1) & 1)
            base = (w & 1) * (2 * WIN)

            def scan_body(i, o):
                sv = mlist[pl.ds(base + i * 16, 16)]
                dv = mlist[pl.ds(base + WIN + i * 16, 16)]
                dl = dv - g0
                msk = (dl >= 0) & (dl < TPR)
                p = (sv << 9) | jnp.clip(dl, 0, TPR - 1)
                plsc.store_compressed(packedE.at[pl.ds(o, 16)], p, mask=msk)
                return o + jnp.sum(jnp.where(msk, 1, 0))

            off = lax.fori_loop(0, WIN // 16, scan_body, off)
            return w + 1, off

        stage(wstart, wstart & 1)
        wnext, off = lax.while_loop(scan_cond, scan_window, (wstart, off0))
        drain_stage()

        packedE[pl.ds(off, 16)] = jnp.full((16,), TPR, jnp.int32)
        ngrp = (off + 15) // 16

        def issue(cidx, buf, sem):
            pltpu.make_async_copy(
                whp_hbm.at[pl.ds(cidx * CHK, CHK), :], buf, sem).start()

        def wait(buf, sem):
            pltpu.make_async_copy(
                whp_hbm.at[pl.ds(0, CHK), :], buf, sem).wait()

        def process(cidx, buf):
            c0 = cidx * CHK
            pltpu.sync_copy(s_hbm.at[pl.ds(c0, CHK)],
                            fbuf.at[pl.ds(SCH, CHK)])

            def rescan(i, mo):
                p = packedE[pl.ds(i * 16, 16)]
                soff = lax.shift_right_logical(p, 9) - c0
                msk = (soff >= 0) & (soff < CHK)
                plsc.store_compressed(mlist.at[pl.ds(mo, 16)], p, mask=msk)
                return mo + jnp.sum(jnp.where(msk, 1, 0))

            moff = lax.fori_loop(0, ngrp, rescan, 0)
            mlist[pl.ds(moff, 16)] = jnp.full((16,), (c0 << 9) | TPR,
                                              jnp.int32)

            def group(gi, _):
                p = mlist[pl.ds(gi * 16, 16)]
                soff = lax.shift_right_logical(p, 9) - c0
                dl = p & (512 - 1)
                sval = plsc.load_gather(fbuf, [soff + SCH])
                dval = plsc.load_gather(fbuf, [dl])
                e = sval + dval
                e = jnp.where(e >= 0.0, e, ALPHA * e)
                w = jnp.exp(e)
                fbuf[pl.ds(WKO, 16)] = w
                plsc.addupdate_scatter(fbuf, [dl + DEN], w)
                mlist[pl.ds(IKO, 16)] = soff
                mlist[pl.ds(IKO + 16, 16)] = dl

                def edge(r, _):
                    rsplat = jnp.full((16,), r, jnp.int32)
                    wsp = plsc.load_gather(fbuf, [rsplat + WKO])
                    ssp = plsc.load_gather(mlist, [rsplat + IKO])
                    dsp = plsc.load_gather(mlist, [rsplat + IKO + 16])

                    @plsc.parallel_loop(0, HW // 16, unroll=HW // 16)
                    def _(k):
                        u = plsc.load_gather(buf, [ssp, lane + 16 * k])
                        ab = plsc.bitcast(u, jnp.bfloat16)
                        a, b = plsc.unpack(
                            ab, format=plsc.PackFormat.INTERLEAVED)
                        ca = lane + 16 * k
                        plsc.addupdate_scatter(hacc, [dsp, ca], wsp * a)
                        plsc.addupdate_scatter(hacc, [dsp, ca + 128], wsp * b)
                    return 0

                lax.fori_loop(0, 16, edge, 0)
                return 0

            lax.fori_loop(0, (moff + 15) // 16, group, 0)

        issue(0, whb0, sem0)
        issue(1, whb1, sem1)

        def pair(pi, _):
            wait(whb0, sem0)
            process(2 * pi, whb0)

            @pl.when(2 * pi + 2 < NCHK)
            def _():
                issue(2 * pi + 2, whb0, sem0)

            wait(whb1, sem1)
            process(2 * pi + 1, whb1)

            @pl.when(2 * pi + 3 < NCHK)
            def _():
                issue(2 * pi + 3, whb1, sem1)
            return 0

        lax.fori_loop(0, NCHK // 2, pair, 0)
        return wnext

    lax.while_loop(lambda w: w < NWIN, round_body, 0)

    def recip(i, _):
        fbuf[pl.ds(DEN + 16 * i, 16)] = 1.0 / fbuf[pl.ds(DEN + 16 * i, 16)]
        return 0
    lax.fori_loop(0, TPR // 16, recip, 0)

    def fin_row(r, _):
        inv = plsc.load_gather(fbuf, [jnp.full((16,), DEN + r, jnp.int32)])
        for k in range(F // 16):
            h = hacc[r, pl.ds(16 * k, 16)] * inv
            hacc[r, pl.ds(16 * k, 16)] = jnp.where(
                h > 0.0, h, jnp.exp(h) - 1.0)
        return 0
    lax.fori_loop(0, TPR, fin_row, 0)

    @pl.when(g0 + TPR <= N_NODES)
    def _():
        pltpu.sync_copy(hacc.at[pl.ds(0, TPR), :], out_hbm.at[pl.ds(g0, TPR)])

    @pl.when((g0 < N_NODES) & (g0 + TPR > N_NODES))
    def _():
        pltpu.sync_copy(hacc.at[pl.ds(0, N_NODES % TPR), :],
                        out_hbm.at[pl.ds(g0, N_NODES % TPR)])


@jax.jit
def kernel(x, edge_index, W, a):
    x_pad = jnp.pad(x, ((0, N_PAD - N_NODES), (0, 0)))
    a2 = a[:, 0].reshape(2, F)
    a8 = jnp.concatenate([a2, jnp.zeros((6, F), jnp.float32)], axis=0)

    wh, sd = pl.pallas_call(
        _tc_body,
        grid=(N_PAD // ROW_BLK,),
        in_specs=[
            pl.BlockSpec((ROW_BLK, F), lambda i: (i, 0)),
            pl.BlockSpec((F, F), lambda i: (0, 0)),
            pl.BlockSpec((8, F), lambda i: (0, 0)),
        ],
        out_specs=[
            pl.BlockSpec((ROW_BLK, F), lambda i: (i, 0)),
            pl.BlockSpec((8, ROW_BLK), lambda i: (0, i)),
        ],
        out_shape=[
            jax.ShapeDtypeStruct((N_PAD, F), jnp.float32),
            jax.ShapeDtypeStruct((8, N_PAD), jnp.float32),
        ],
    )(x_pad, W, a8)

    whbf = wh.astype(jnp.bfloat16)
    whp = jax.lax.bitcast_convert_type(
        jnp.stack([whbf[:, :HW], whbf[:, HW:]], axis=-1), jnp.int32)

    src = edge_index[0].astype(jnp.int32)
    dst = edge_index[1].astype(jnp.int32)
    src = jnp.concatenate([src, jnp.zeros((E_PAD - N_EDGES,), jnp.int32)])
    dst = jnp.concatenate(
        [dst, jnp.full((E_PAD - N_EDGES,), -1, jnp.int32)])

    sc = pl.kernel(
        _sc_body,
        out_type=jax.ShapeDtypeStruct((N_NODES, F), jnp.float32),
        mesh=plsc.VectorSubcoreMesh(core_axis_name="c", subcore_axis_name="s"),
        compiler_params=pltpu.CompilerParams(needs_layout_passes=False),
        scratch_types=[
            pltpu.VMEM((TPR + 1, F), jnp.float32),
            pltpu.VMEM((CAPE + 16,), jnp.int32),
            pltpu.VMEM((CAPE + 48,), jnp.int32),
            pltpu.VMEM((CHK, HW), jnp.int32),
            pltpu.VMEM((CHK, HW), jnp.int32),
            pltpu.VMEM((816,), jnp.float32),
            pltpu.SemaphoreType.DMA,
            pltpu.SemaphoreType.DMA,
            pltpu.SemaphoreType.DMA,
        ],
    )
    return sc(whp, sd[0], sd[1], src, dst)
